# baseline (speedup 1.0000x reference)
import functools

import jax
import jax.numpy as jnp
from jax import lax
from jax.experimental import pallas as pl
from jax.experimental.pallas import tpu as pltpu
from jax.experimental.pallas import tpu_sc as plsc

D = 32
H1 = 64
H2 = 32
NC = 2
NS = 16
NW = NC * NS
CHUNK = 128
PK = 128 // D


@functools.lru_cache(maxsize=None)
def _make_sc_gather(batch: int, rows_packed: int):
    bpw = batch // NW
    nchunk = bpw // CHUNK
    mesh = plsc.VectorSubcoreMesh(core_axis_name="c", subcore_axis_name="s")

    @functools.partial(
        pl.kernel,
        mesh=mesh,
        out_type=(
            jax.ShapeDtypeStruct((batch, 128), jnp.float32),
            jax.ShapeDtypeStruct((batch, 128), jnp.float32),
        ),
        scratch_types=[
            pltpu.VMEM((bpw,), jnp.int32),
            pltpu.VMEM((bpw,), jnp.int32),
            pltpu.VMEM((bpw, 128), jnp.float32),
            pltpu.SemaphoreType.DMA,
        ],
    )
    def gather_kernel(urow_hbm, irow_hbm, utab_hbm, itab_hbm,
                      uout_hbm, iout_hbm,
                      uidx_v, iidx_v, rows_v, sem):
        wid = lax.axis_index("s") * NC + lax.axis_index("c")
        base = wid * bpw
        pltpu.sync_copy(urow_hbm.at[pl.ds(base, bpw)], uidx_v)
        pltpu.sync_copy(irow_hbm.at[pl.ds(base, bpw)], iidx_v)
        copies = []
        for j in range(nchunk):
            sl = pl.ds(j * CHUNK, CHUNK)
            copies.append(
                pltpu.async_copy(utab_hbm.at[uidx_v.at[sl]], rows_v.at[sl], sem))
        for c in copies:
            c.wait()
        pltpu.sync_copy(rows_v, uout_hbm.at[pl.ds(base, bpw)])
        copies = []
        for j in range(nchunk):
            sl = pl.ds(j * CHUNK, CHUNK)
            copies.append(
                pltpu.async_copy(itab_hbm.at[iidx_v.at[sl]], rows_v.at[sl], sem))
        for c in copies:
            c.wait()
        pltpu.sync_copy(rows_v, iout_hbm.at[pl.ds(base, bpw)])

    return gather_kernel


def _mlp_body(u_ref, i_ref, qu_ref, qi_ref, w1a_ref, w1b_ref, b1_ref,
              w2_ref, b2_ref, w3_ref, b3_ref, o_ref):
    blk = u_ref.shape[0]
    colgrp = lax.broadcasted_iota(jnp.int32, (blk, 128), 1) // D
    um = jnp.where(colgrp == qu_ref[...], u_ref[...], 0.0)
    im = jnp.where(colgrp == qi_ref[...], i_ref[...], 0.0)
    h = jnp.dot(um, w1a_ref[...], preferred_element_type=jnp.float32)
    h = h + jnp.dot(im, w1b_ref[...], preferred_element_type=jnp.float32)
    h = jnp.maximum(h + b1_ref[...], 0.0)
    h = jnp.dot(h, w2_ref[...], preferred_element_type=jnp.float32)
    h = jnp.maximum(h + b2_ref[...], 0.0)
    o_ref[...] = (jnp.dot(h, w3_ref[...], preferred_element_type=jnp.float32)
                  + b3_ref[...])


@functools.lru_cache(maxsize=None)
def _make_tc_mlp(batch: int, blk: int):
    grid = (batch // blk,)
    row_spec = pl.BlockSpec((blk, 128), lambda i: (i, 0))
    q_spec = pl.BlockSpec((blk, 1), lambda i: (i, 0))
    full = lambda shape: pl.BlockSpec(shape, lambda i: (0, 0))
    return pl.pallas_call(
        _mlp_body,
        grid=grid,
        in_specs=[
            row_spec,
            row_spec,
            q_spec,
            q_spec,
            full((128, H1)),
            full((128, H1)),
            full((1, H1)),
            full((H1, H2)),
            full((1, H2)),
            full((H2, 1)),
            full((1, 1)),
        ],
        out_specs=pl.BlockSpec((blk, 1), lambda i: (i, 0)),
        out_shape=jax.ShapeDtypeStruct((batch, 1), jnp.float32),
    )


def kernel(user_indices, item_indices, user_table, item_table,
           W1, b1, W2, b2, W3, b3):
    batch = user_indices.shape[0]
    nu, nd = user_table.shape
    rows_packed = nu * nd // 128
    ut2 = jnp.reshape(user_table, (rows_packed, 128))
    it2 = jnp.reshape(item_table, (rows_packed, 128))
    uidx = user_indices.astype(jnp.int32)
    iidx = item_indices.astype(jnp.int32)
    u128, i128 = _make_sc_gather(batch, rows_packed)(
        uidx // PK, iidx // PK, ut2, it2)
    qu = (uidx % PK).reshape(batch, 1)
    qi = (iidx % PK).reshape(batch, 1)
    w1a4 = jnp.concatenate([W1[:D]] * PK, axis=0)
    w1b4 = jnp.concatenate([W1[D:]] * PK, axis=0)
    blk = 2048 if batch % 2048 == 0 else batch
    mlp = _make_tc_mlp(batch, blk)
    return mlp(u128, i128, qu, qi, w1a4, w1b4, b1.reshape(1, H1),
               W2, b2.reshape(1, H2), W3, b3.reshape(1, 1))
